# baseline (speedup 1.0000x reference)
import functools

import jax
import jax.numpy as jnp
from jax import lax
from jax.experimental import pallas as pl
from jax.experimental.pallas import tpu as pltpu
from jax.experimental.pallas import tpu_sc as plsc

TAU = 2.0
V_TH = 1.0
BN_EPS = 1e-5

_PREC = jax.lax.Precision.HIGHEST


def _lif_unrolled(hs):
    v = jnp.zeros_like(hs[0])
    spikes = []
    for h in hs:
        v = v + (h - v) / TAU
        s = (v >= V_TH).astype(hs[0].dtype)
        v = (1.0 - s) * v
        spikes.append(s)
    return spikes


def _gate_body(kv_ref, w_ref, gamma_ref, beta_ref, mean_ref, var_ref, g_ref):
    T = kv_ref.shape[0]
    scale = gamma_ref[...] * jax.lax.rsqrt(var_ref[...] + BN_EPS)
    shift = beta_ref[...] - mean_ref[...] * scale
    hs = []
    for t in range(T):
        h = jax.lax.dot_general(kv_ref[t], w_ref[...], (((1,), (1,)), ((), ())),
                                preferred_element_type=jnp.float32,
                                precision=_PREC)
        hs.append(h * scale + shift)
    spikes = _lif_unrolled(hs)
    g_ref[...] = sum(spikes) / float(T)


_LW = 128


def _select_body(q_ref, g_ref, i1_ref, i2_ref, v1_ref, v2_ref, *, nkv, sub):
    T, blk, D = q_ref.shape
    sscale = float(D) ** (-0.5)
    nch = nkv // _LW
    neg = jnp.float32(-3.4e38)
    lane = jax.lax.broadcasted_iota(jnp.int32, (sub, _LW), 1)
    for t in range(T):
        s = jax.lax.dot_general(q_ref[t], g_ref[...], (((1,), (1,)), ((), ())),
                                preferred_element_type=jnp.float32,
                                precision=_PREC) * sscale
        for u in range(blk // sub):
            r0 = u * sub
            M = jax.lax.slice(s, (r0, 0), (r0 + sub, _LW))
            C = jnp.zeros((sub, _LW), jnp.int32)
            M2 = jnp.full((sub, _LW), neg, jnp.float32)
            C2 = jnp.zeros((sub, _LW), jnp.int32)
            for c in range(1, nch):
                sc = jax.lax.slice(s, (r0, c * _LW), (r0 + sub, (c + 1) * _LW))
                cond1 = sc > M
                cond2 = sc > M2
                C2 = jnp.where(cond1, C, jnp.where(cond2, c, C2))
                M2 = jnp.where(cond1, M, jnp.where(cond2, sc, M2))
                C = jnp.where(cond1, c, C)
                M = jnp.maximum(M, sc)
            gidx = C * _LW + lane
            m1 = jnp.max(M, axis=1, keepdims=True)
            i1 = jnp.min(jnp.where(M == m1, gidx, nkv), axis=1, keepdims=True)
            l1 = jax.lax.bitwise_and(i1, _LW - 1)
            Mp = jnp.where(lane == l1, M2, M)
            Cp = jnp.where(lane == l1, C2, C)
            m2 = jnp.max(Mp, axis=1, keepdims=True)
            gidx2 = Cp * _LW + lane
            i2 = jnp.min(jnp.where(Mp == m2, gidx2, nkv), axis=1, keepdims=True)
            rows = pl.ds(r0, sub)
            i1_ref[rows, pl.ds(t, 1)] = i1
            i2_ref[rows, pl.ds(t, 1)] = i2
            v1_ref[rows, pl.ds(t, 1)] = m1
            v2_ref[rows, pl.ds(t, 1)] = m2


def _proj_body(g1_ref, g2_ref, v1_ref, v2_ref, w_ref, gamma_ref, beta_ref,
               mean_ref, var_ref, out_ref):
    blk, T, D = g1_ref.shape
    scale = gamma_ref[...] * jax.lax.rsqrt(var_ref[...] + BN_EPS)
    shift = beta_ref[...] - mean_ref[...] * scale
    hs = []
    for t in range(T):
        u = (g1_ref[:, t, :] * v1_ref[:, pl.ds(t, 1)]
             + g2_ref[:, t, :] * v2_ref[:, pl.ds(t, 1)])
        h = jax.lax.dot_general(u, w_ref[...],
                                (((1,), (1,)), ((), ())),
                                preferred_element_type=jnp.float32,
                                precision=_PREC)
        hs.append(h * scale + shift)
    spikes = _lif_unrolled(hs)
    for t in range(T):
        out_ref[t] = spikes[t]


def _sc_gather2(g, i1, i2):
    (R,) = i1.shape
    D = g.shape[1]
    NC, NS, L = 2, 16, 16
    NW = NC * NS
    rpw = R // NW
    CH = 128
    mesh = plsc.VectorSubcoreMesh(core_axis_name="c", subcore_axis_name="s")

    @functools.partial(
        pl.kernel, mesh=mesh,
        out_type=(jax.ShapeDtypeStruct((R, D), jnp.float32),
                  jax.ShapeDtypeStruct((R, D), jnp.float32)),
        scratch_types=[
            pltpu.VMEM((CH,), jnp.int32),
            pltpu.VMEM((CH,), jnp.int32),
            pltpu.VMEM((CH, D), jnp.float32),
            pltpu.VMEM((CH, D), jnp.float32),
            pltpu.SemaphoreType.DMA,
            pltpu.SemaphoreType.DMA,
        ],
    )
    def sc_kernel(g_hbm, i1_hbm, i2_hbm, o1_hbm, o2_hbm,
                  idx1_v, idx2_v, rows1_v, rows2_v, sem1, sem2):
        wid = lax.axis_index("s") * NC + lax.axis_index("c")
        for c in range(rpw // CH):
            base = wid * rpw + c * CH
            pltpu.sync_copy(i1_hbm.at[pl.ds(base, CH)], idx1_v)
            pltpu.sync_copy(i2_hbm.at[pl.ds(base, CH)], idx2_v)
            cp1 = pltpu.async_copy(g_hbm.at[idx1_v], rows1_v, sem1)
            cp2 = pltpu.async_copy(g_hbm.at[idx2_v], rows2_v, sem2)
            cp1.wait()
            cp2.wait()
            pltpu.sync_copy(rows1_v, o1_hbm.at[pl.ds(base, CH)])
            pltpu.sync_copy(rows2_v, o2_hbm.at[pl.ds(base, CH)])

    return sc_kernel(g, i1, i2)


def kernel(q, kv, gate_W, gate_gamma, gate_beta, gate_mean, gate_var,
           proj_W, proj_gamma, proj_beta, proj_mean, proj_var):
    T, B, Nq, D = q.shape
    Nkv = kv.shape[2]
    kv3 = kv.reshape(T, B * Nkv, D)
    q3 = q.reshape(T, B * Nq, D)
    row = lambda a: a.reshape(1, D)

    blk_g = 512
    g = pl.pallas_call(
        _gate_body,
        grid=(Nkv // blk_g,),
        in_specs=[
            pl.BlockSpec((T, blk_g, D), lambda i: (0, i, 0)),
            pl.BlockSpec((D, D), lambda i: (0, 0)),
            pl.BlockSpec((1, D), lambda i: (0, 0)),
            pl.BlockSpec((1, D), lambda i: (0, 0)),
            pl.BlockSpec((1, D), lambda i: (0, 0)),
            pl.BlockSpec((1, D), lambda i: (0, 0)),
        ],
        out_specs=pl.BlockSpec((blk_g, D), lambda i: (i, 0)),
        out_shape=jax.ShapeDtypeStruct((Nkv, D), jnp.float32),
    )(kv3, gate_W, row(gate_gamma), row(gate_beta), row(gate_mean),
      row(gate_var))

    blk_q = 1024
    i1, i2, v1, v2 = pl.pallas_call(
        functools.partial(_select_body, nkv=Nkv, sub=64),
        grid=(Nq // blk_q,),
        in_specs=[
            pl.BlockSpec((T, blk_q, D), lambda i: (0, i, 0)),
            pl.BlockSpec((Nkv, D), lambda i: (0, 0)),
        ],
        out_specs=[
            pl.BlockSpec((blk_q, T), lambda i: (i, 0)),
            pl.BlockSpec((blk_q, T), lambda i: (i, 0)),
            pl.BlockSpec((blk_q, T), lambda i: (i, 0)),
            pl.BlockSpec((blk_q, T), lambda i: (i, 0)),
        ],
        out_shape=[
            jax.ShapeDtypeStruct((Nq, T), jnp.int32),
            jax.ShapeDtypeStruct((Nq, T), jnp.int32),
            jax.ShapeDtypeStruct((Nq, T), jnp.float32),
            jax.ShapeDtypeStruct((Nq, T), jnp.float32),
        ],
    )(q3, g)

    NP = 1
    Nqp = Nq // NP
    i1f, i2f = i1.reshape(-1), i2.reshape(-1)
    outs = []
    for p in range(NP):
        rs = slice(p * Nqp * T, (p + 1) * Nqp * T)
        qs = slice(p * Nqp, (p + 1) * Nqp)
        g1, g2 = _sc_gather2(g, i1f[rs], i2f[rs])
        outs.append(pl.pallas_call(
            _proj_body,
            grid=(Nqp // blk_q,),
            in_specs=[
                pl.BlockSpec((blk_q, T, D), lambda i: (i, 0, 0)),
                pl.BlockSpec((blk_q, T, D), lambda i: (i, 0, 0)),
                pl.BlockSpec((blk_q, T), lambda i: (i, 0)),
                pl.BlockSpec((blk_q, T), lambda i: (i, 0)),
                pl.BlockSpec((D, D), lambda i: (0, 0)),
                pl.BlockSpec((1, D), lambda i: (0, 0)),
                pl.BlockSpec((1, D), lambda i: (0, 0)),
                pl.BlockSpec((1, D), lambda i: (0, 0)),
                pl.BlockSpec((1, D), lambda i: (0, 0)),
            ],
            out_specs=pl.BlockSpec((T, blk_q, D), lambda i: (0, i, 0)),
            out_shape=jax.ShapeDtypeStruct((T, Nqp, D), jnp.float32),
        )(g1.reshape(Nqp, T, D), g2.reshape(Nqp, T, D), v1[qs], v2[qs],
          proj_W, row(proj_gamma), row(proj_beta), row(proj_mean),
          row(proj_var)))

    out = jnp.concatenate(outs, axis=1)
    return out.reshape(T, B, Nq, D)


# speedup vs baseline: 1.1344x; 1.1344x over previous
import functools

import jax
import jax.numpy as jnp
from jax import lax
from jax.experimental import pallas as pl
from jax.experimental.pallas import tpu as pltpu
from jax.experimental.pallas import tpu_sc as plsc

TAU = 2.0
V_TH = 1.0
BN_EPS = 1e-5

_PREC = jax.lax.Precision.HIGHEST


def _lif_unrolled(hs):
    v = jnp.zeros_like(hs[0])
    spikes = []
    for h in hs:
        v = v + (h - v) / TAU
        s = (v >= V_TH).astype(hs[0].dtype)
        v = (1.0 - s) * v
        spikes.append(s)
    return spikes


def _gate_body(kv_ref, w_ref, gamma_ref, beta_ref, mean_ref, var_ref, g_ref):
    T = kv_ref.shape[0]
    scale = gamma_ref[...] * jax.lax.rsqrt(var_ref[...] + BN_EPS)
    shift = beta_ref[...] - mean_ref[...] * scale
    hs = []
    for t in range(T):
        h = jax.lax.dot_general(kv_ref[t], w_ref[...], (((1,), (1,)), ((), ())),
                                preferred_element_type=jnp.float32,
                                precision=_PREC)
        hs.append(h * scale + shift)
    spikes = _lif_unrolled(hs)
    g_ref[...] = sum(spikes) / float(T)


_LW = 128


def _select_body(q_ref, g_ref, i1_ref, i2_ref, v1_ref, v2_ref, *, nkv, sub):
    T, blk, D = q_ref.shape
    sscale = float(D) ** (-0.5)
    nch = nkv // _LW
    neg = jnp.float32(-3.4e38)
    lane = jax.lax.broadcasted_iota(jnp.int32, (sub, _LW), 1)
    for t in range(T):
        s = jax.lax.dot_general(q_ref[t], g_ref[...], (((1,), (1,)), ((), ())),
                                preferred_element_type=jnp.float32,
                                precision=_PREC) * sscale
        for u in range(blk // sub):
            r0 = u * sub
            M = jax.lax.slice(s, (r0, 0), (r0 + sub, _LW))
            C = jnp.zeros((sub, _LW), jnp.int32)
            M2 = jnp.full((sub, _LW), neg, jnp.float32)
            C2 = jnp.zeros((sub, _LW), jnp.int32)
            for c in range(1, nch):
                sc = jax.lax.slice(s, (r0, c * _LW), (r0 + sub, (c + 1) * _LW))
                cond1 = sc > M
                cond2 = sc > M2
                C2 = jnp.where(cond1, C, jnp.where(cond2, c, C2))
                M2 = jnp.where(cond1, M, jnp.where(cond2, sc, M2))
                C = jnp.where(cond1, c, C)
                M = jnp.maximum(M, sc)
            gidx = C * _LW + lane
            m1 = jnp.max(M, axis=1, keepdims=True)
            i1 = jnp.min(jnp.where(M == m1, gidx, nkv), axis=1, keepdims=True)
            l1 = jax.lax.bitwise_and(i1, _LW - 1)
            Mp = jnp.where(lane == l1, M2, M)
            Cp = jnp.where(lane == l1, C2, C)
            m2 = jnp.max(Mp, axis=1, keepdims=True)
            gidx2 = Cp * _LW + lane
            i2 = jnp.min(jnp.where(Mp == m2, gidx2, nkv), axis=1, keepdims=True)
            rows = pl.ds(r0, sub)
            i1_ref[rows, pl.ds(t, 1)] = i1
            i2_ref[rows, pl.ds(t, 1)] = i2
            v1_ref[rows, pl.ds(t, 1)] = m1
            v2_ref[rows, pl.ds(t, 1)] = m2


def _proj_body(g1_ref, g2_ref, v1_ref, v2_ref, w_ref, gamma_ref, beta_ref,
               mean_ref, var_ref, out_ref):
    blk, T, D = g1_ref.shape
    scale = gamma_ref[...] * jax.lax.rsqrt(var_ref[...] + BN_EPS)
    shift = beta_ref[...] - mean_ref[...] * scale
    hs = []
    for t in range(T):
        u = (g1_ref[:, t, :] * v1_ref[:, pl.ds(t, 1)]
             + g2_ref[:, t, :] * v2_ref[:, pl.ds(t, 1)])
        h = jax.lax.dot_general(u, w_ref[...],
                                (((1,), (1,)), ((), ())),
                                preferred_element_type=jnp.float32,
                                precision=_PREC)
        hs.append(h * scale + shift)
    spikes = _lif_unrolled(hs)
    for t in range(T):
        out_ref[t] = spikes[t]


def _sc_gather2(g, i1, i2):
    (R,) = i1.shape
    D = g.shape[1]
    NC, NS, L = 2, 16, 16
    NW = NC * NS
    rpw = R // NW
    CH = 128
    mesh = plsc.VectorSubcoreMesh(core_axis_name="c", subcore_axis_name="s")

    @functools.partial(
        pl.kernel, mesh=mesh,
        out_type=(jax.ShapeDtypeStruct((R, D), jnp.float32),
                  jax.ShapeDtypeStruct((R, D), jnp.float32)),
        scratch_types=[
            pltpu.VMEM((CH,), jnp.int32),
            pltpu.VMEM((CH,), jnp.int32),
            pltpu.VMEM((CH, D), jnp.float32),
            pltpu.VMEM((CH, D), jnp.float32),
            pltpu.SemaphoreType.DMA,
            pltpu.SemaphoreType.DMA,
        ],
    )
    def sc_kernel(g_hbm, i1_hbm, i2_hbm, o1_hbm, o2_hbm,
                  idx1_v, idx2_v, rows1_v, rows2_v, sem1, sem2):
        wid = lax.axis_index("s") * NC + lax.axis_index("c")
        for c in range(rpw // CH):
            base = wid * rpw + c * CH
            pltpu.sync_copy(i1_hbm.at[pl.ds(base, CH)], idx1_v)
            pltpu.sync_copy(i2_hbm.at[pl.ds(base, CH)], idx2_v)
            cp1 = pltpu.async_copy(g_hbm.at[idx1_v], rows1_v, sem1)
            cp2 = pltpu.async_copy(g_hbm.at[idx2_v], rows2_v, sem2)
            cp1.wait()
            cp2.wait()
            pltpu.sync_copy(rows1_v, o1_hbm.at[pl.ds(base, CH)])
            pltpu.sync_copy(rows2_v, o2_hbm.at[pl.ds(base, CH)])

    return sc_kernel(g, i1, i2)


def kernel(q, kv, gate_W, gate_gamma, gate_beta, gate_mean, gate_var,
           proj_W, proj_gamma, proj_beta, proj_mean, proj_var):
    T, B, Nq, D = q.shape
    Nkv = kv.shape[2]
    kv3 = kv.reshape(T, B * Nkv, D)
    q3 = q.reshape(T, B * Nq, D)
    row = lambda a: a.reshape(1, D)

    blk_g = 512
    g = pl.pallas_call(
        _gate_body,
        grid=(Nkv // blk_g,),
        in_specs=[
            pl.BlockSpec((T, blk_g, D), lambda i: (0, i, 0)),
            pl.BlockSpec((D, D), lambda i: (0, 0)),
            pl.BlockSpec((1, D), lambda i: (0, 0)),
            pl.BlockSpec((1, D), lambda i: (0, 0)),
            pl.BlockSpec((1, D), lambda i: (0, 0)),
            pl.BlockSpec((1, D), lambda i: (0, 0)),
        ],
        out_specs=pl.BlockSpec((blk_g, D), lambda i: (i, 0)),
        out_shape=jax.ShapeDtypeStruct((Nkv, D), jnp.float32),
    )(kv3, gate_W, row(gate_gamma), row(gate_beta), row(gate_mean),
      row(gate_var))

    blk_q = 512
    i1, i2, v1, v2 = pl.pallas_call(
        functools.partial(_select_body, nkv=Nkv, sub=64),
        grid=(Nq // blk_q,),
        in_specs=[
            pl.BlockSpec((T, blk_q, D), lambda i: (0, i, 0)),
            pl.BlockSpec((Nkv, D), lambda i: (0, 0)),
        ],
        out_specs=[
            pl.BlockSpec((blk_q, T), lambda i: (i, 0)),
            pl.BlockSpec((blk_q, T), lambda i: (i, 0)),
            pl.BlockSpec((blk_q, T), lambda i: (i, 0)),
            pl.BlockSpec((blk_q, T), lambda i: (i, 0)),
        ],
        out_shape=[
            jax.ShapeDtypeStruct((Nq, T), jnp.int32),
            jax.ShapeDtypeStruct((Nq, T), jnp.int32),
            jax.ShapeDtypeStruct((Nq, T), jnp.float32),
            jax.ShapeDtypeStruct((Nq, T), jnp.float32),
        ],
    )(q3, g)

    NP = 1
    Nqp = Nq // NP
    i1f, i2f = i1.reshape(-1), i2.reshape(-1)
    outs = []
    for p in range(NP):
        rs = slice(p * Nqp * T, (p + 1) * Nqp * T)
        qs = slice(p * Nqp, (p + 1) * Nqp)
        g1, g2 = _sc_gather2(g, i1f[rs], i2f[rs])
        outs.append(pl.pallas_call(
            _proj_body,
            grid=(Nqp // blk_q,),
            in_specs=[
                pl.BlockSpec((blk_q, T, D), lambda i: (i, 0, 0)),
                pl.BlockSpec((blk_q, T, D), lambda i: (i, 0, 0)),
                pl.BlockSpec((blk_q, T), lambda i: (i, 0)),
                pl.BlockSpec((blk_q, T), lambda i: (i, 0)),
                pl.BlockSpec((D, D), lambda i: (0, 0)),
                pl.BlockSpec((1, D), lambda i: (0, 0)),
                pl.BlockSpec((1, D), lambda i: (0, 0)),
                pl.BlockSpec((1, D), lambda i: (0, 0)),
                pl.BlockSpec((1, D), lambda i: (0, 0)),
            ],
            out_specs=pl.BlockSpec((T, blk_q, D), lambda i: (0, i, 0)),
            out_shape=jax.ShapeDtypeStruct((T, Nqp, D), jnp.float32),
        )(g1.reshape(Nqp, T, D), g2.reshape(Nqp, T, D), v1[qs], v2[qs],
          proj_W, row(proj_gamma), row(proj_beta), row(proj_mean),
          row(proj_var)))

    out = jnp.concatenate(outs, axis=1)
    return out.reshape(T, B, Nq, D)
